# Initial kernel scaffold; baseline (speedup 1.0000x reference)
import jax
import jax.numpy as jnp
from jax.experimental import pallas as pl


def kernel(x, edge_index, batch, W1, as1, ad1, b1, W2, as2, ad2, b2, W3, as3, ad3, b3, fcW, fcb):
    raise NotImplementedError("write your pallas kernel here")


# speedup vs baseline: 1.0008x; 1.0008x over previous
import functools

import jax
import jax.numpy as jnp
from jax.experimental import pallas as pl
from jax.experimental.pallas import tpu as pltpu

N = 10000
E = 320000
D = 128
H = 8
G = 64
CLS = 10


def _gat_jax(x, src, dst, W, att_src, att_dst, bias, heads, out_ch, concat):
    n = x.shape[0]
    xp = (x @ W).reshape(n, heads, out_ch)
    a_src = (xp * att_src).sum(-1)
    a_dst = (xp * att_dst).sum(-1)
    alpha = a_src[src] + a_dst[dst]
    alpha = jax.nn.leaky_relu(alpha, negative_slope=0.2)
    amax = jax.ops.segment_max(alpha, dst, num_segments=n)
    amax = jnp.where(jnp.isfinite(amax), amax, 0.0)
    ex = jnp.exp(alpha - amax[dst])
    denom = jax.ops.segment_sum(ex, dst, num_segments=n)
    coef = ex / (denom[dst] + 1e-16)
    msg = xp[src] * coef[..., None]
    out = jax.ops.segment_sum(msg, dst, num_segments=n)
    if concat:
        out = out.reshape(n, heads * out_ch)
    else:
        out = out.mean(axis=1)
    return out + bias


def _final_body(h3_ref, batch_ref, fcW_ref, fcb_ref, out_ref):
    h3 = h3_ref[...]
    batch = batch_ref[0, :]
    gids = jax.lax.broadcasted_iota(jnp.int32, (G, N), 0)
    oh = (batch[None, :] == gids).astype(jnp.float32)
    pooled = jnp.dot(oh, h3, preferred_element_type=jnp.float32)
    counts = jnp.sum(oh, axis=1, keepdims=True)
    pooled = pooled / jnp.maximum(counts, 1.0)
    logits = (jnp.dot(pooled, fcW_ref[...], preferred_element_type=jnp.float32)
              + fcb_ref[0, :])
    out_ref[...] = jax.nn.log_softmax(logits, axis=1)


@jax.jit
def _final_stage(h3, batch, fcW, fcb):
    return pl.pallas_call(
        _final_body,
        out_shape=jax.ShapeDtypeStruct((G, CLS), jnp.float32),
    )(h3, batch.reshape(1, N).astype(jnp.int32), fcW, fcb.reshape(1, CLS))


def kernel(x, edge_index, batch, W1, as1, ad1, b1, W2, as2, ad2, b2,
           W3, as3, ad3, b3, fcW, fcb):
    src = edge_index[0]
    dst = edge_index[1]
    h = x.astype(jnp.float32)
    h = jax.nn.elu(_gat_jax(h, src, dst, W1, as1, ad1, b1, H, 8, True))
    h = jax.nn.elu(_gat_jax(h, src, dst, W2, as2, ad2, b2, H, 8, True))
    h = jax.nn.elu(_gat_jax(h, src, dst, W3, as3, ad3, b3, H, 64, False))
    return _final_stage(h, batch, fcW, fcb)


# speedup vs baseline: 39.8611x; 39.8299x over previous
import dataclasses
import functools

import jax
import jax.numpy as jnp
from jax import lax
from jax.experimental import pallas as pl
from jax.experimental.pallas import tpu as pltpu
from jax.experimental.pallas import tpu_sc as plsc

N = 10000
E = 320000
D = 128
H = 8
G = 64
CLS = 10

NP = 10240
BLK = 2048
NBLK = NP // BLK
CE = 128
EPAD = E + 16 * CE
NEG = -1e30


def _cmax_update(c_ref, att, i):
    bm = jnp.max(att, axis=0, keepdims=True)
    prev = jnp.where(i == 0, jnp.full((1, 16), NEG, jnp.float32), c_ref[...])
    cur = jnp.maximum(prev, bm)

    @pl.when(i < NBLK - 1)
    def _():
        c_ref[...] = cur

    @pl.when(i == NBLK - 1)
    def _():
        c_ref[...] = jnp.concatenate(
            [cur[:, :8] + cur[:, 8:], jnp.zeros((1, 8), jnp.float32)], axis=1)


def _feat_pack(feat, att):
    return jnp.concatenate(
        [feat, att[:, :8], jnp.zeros((BLK, 56), jnp.float32)], axis=1)


def _att_t(a, feat):
    return lax.dot_general(a, feat, (((0,), (1,)), ((), ())),
                           preferred_element_type=jnp.float32)


def _tc_first_body(x_ref, w_ref, a_ref, fx_ref, att_ref, c_ref):
    i = pl.program_id(0)
    feat = jnp.dot(x_ref[...], w_ref[...], preferred_element_type=jnp.float32)
    att = jnp.dot(feat, a_ref[...], preferred_element_type=jnp.float32)
    fx_ref[...] = _feat_pack(feat, att)
    att_ref[...] = _att_t(a_ref[...], feat)
    _cmax_update(c_ref, att, i)


def _tc_first(x, w, a):
    return pl.pallas_call(
        _tc_first_body,
        grid=(NBLK,),
        in_specs=[
            pl.BlockSpec((BLK, D), lambda i: (i, 0)),
            pl.BlockSpec((D, 64), lambda i: (0, 0)),
            pl.BlockSpec((64, 16), lambda i: (0, 0)),
        ],
        out_specs=[
            pl.BlockSpec((BLK, 128), lambda i: (i, 0)),
            pl.BlockSpec((16, BLK), lambda i: (0, i)),
            pl.BlockSpec((1, 16), lambda i: (0, 0)),
        ],
        out_shape=[
            jax.ShapeDtypeStruct((NP, 128), jnp.float32),
            jax.ShapeDtypeStruct((16, NP), jnp.float32),
            jax.ShapeDtypeStruct((1, 16), jnp.float32),
        ],
    )(x, w, a)


def _tc_mid_body(raw_ref, b_ref, w_ref, a_ref, fx_ref, att_ref, c_ref):
    i = pl.program_id(0)
    rows = jax.lax.broadcasted_iota(jnp.int32, (8, 64), 0)
    cols = jax.lax.broadcasted_iota(jnp.int32, (8, 64), 1)
    r8 = (cols // 8 == rows).astype(jnp.float32)
    raw = raw_ref[...]
    dd = jnp.dot(raw[:, 64:72], r8, preferred_element_type=jnp.float32)
    h = raw[:, :64] / (dd + 1e-16) + b_ref[...]
    h = jnp.where(h > 0, h, jnp.exp(h) - 1.0)
    feat = jnp.dot(h, w_ref[...], preferred_element_type=jnp.float32)
    att = jnp.dot(feat, a_ref[...], preferred_element_type=jnp.float32)
    fx_ref[...] = _feat_pack(feat, att)
    att_ref[...] = _att_t(a_ref[...], feat)
    _cmax_update(c_ref, att, i)


def _tc_mid(raw, b, w, a):
    return pl.pallas_call(
        _tc_mid_body,
        grid=(NBLK,),
        in_specs=[
            pl.BlockSpec((BLK, 128), lambda i: (i, 0)),
            pl.BlockSpec((1, 64), lambda i: (0, 0)),
            pl.BlockSpec((64, 64), lambda i: (0, 0)),
            pl.BlockSpec((64, 16), lambda i: (0, 0)),
        ],
        out_specs=[
            pl.BlockSpec((BLK, 128), lambda i: (i, 0)),
            pl.BlockSpec((16, BLK), lambda i: (0, i)),
            pl.BlockSpec((1, 16), lambda i: (0, 0)),
        ],
        out_shape=[
            jax.ShapeDtypeStruct((NP, 128), jnp.float32),
            jax.ShapeDtypeStruct((16, NP), jnp.float32),
            jax.ShapeDtypeStruct((1, 16), jnp.float32),
        ],
    )(raw, b, w, a)


def _tc_out_body(r0_ref, r1_ref, r2_ref, r3_ref, r4_ref, b_ref, w_ref,
                 batch_ref, fcw_ref, fcb_ref, out_ref, pool_ref):
    i = pl.program_id(0)
    rows = jax.lax.broadcasted_iota(jnp.int32, (8, 512), 0)
    cols = jax.lax.broadcasted_iota(jnp.int32, (8, 512), 1)
    r8 = (cols // 64 == rows).astype(jnp.float32)
    agg = jnp.concatenate(
        [r0_ref[...], r1_ref[...], r2_ref[...], r3_ref[...]], axis=1)
    dd = jnp.dot(r4_ref[...][:, :8], r8, preferred_element_type=jnp.float32)
    h = jnp.dot(agg / (dd + 1e-16), w_ref[...],
                preferred_element_type=jnp.float32) * 0.125 + b_ref[...]
    h = jnp.where(h > 0, h, jnp.exp(h) - 1.0)
    haug = jnp.concatenate([h, jnp.ones((BLK, 1), jnp.float32)], axis=1)
    gids = jax.lax.broadcasted_iota(jnp.int32, (G, BLK), 0)
    oh = (batch_ref[0] == gids).astype(jnp.float32)
    part = jnp.dot(oh, haug, preferred_element_type=jnp.float32)

    @pl.when(i == 0)
    def _():
        pool_ref[:, :65] = part

    @pl.when(i > 0)
    def _():
        pool_ref[:, :65] = pool_ref[:, :65] + part

    @pl.when(i == NBLK - 1)
    def _():
        acc = pool_ref[:, :65]
        pooled = acc[:, :64] / jnp.maximum(acc[:, 64:65], 1.0)
        logits = jnp.dot(pooled, fcw_ref[...],
                         preferred_element_type=jnp.float32) + fcb_ref[...]
        m = jnp.max(logits, axis=1, keepdims=True)
        z = logits - m
        out_ref[...] = z - jnp.log(jnp.sum(jnp.exp(z), axis=1, keepdims=True))


def _tc_out(raws, b, w, batch, fcw, fcb):
    return pl.pallas_call(
        _tc_out_body,
        grid=(NBLK,),
        in_specs=[
            pl.BlockSpec((BLK, 128), lambda i: (i, 0)),
            pl.BlockSpec((BLK, 128), lambda i: (i, 0)),
            pl.BlockSpec((BLK, 128), lambda i: (i, 0)),
            pl.BlockSpec((BLK, 128), lambda i: (i, 0)),
            pl.BlockSpec((BLK, 128), lambda i: (i, 0)),
            pl.BlockSpec((1, 64), lambda i: (0, 0)),
            pl.BlockSpec((512, 64), lambda i: (0, 0)),
            pl.BlockSpec((1, 1, BLK), lambda i: (i, 0, 0)),
            pl.BlockSpec((64, CLS), lambda i: (0, 0)),
            pl.BlockSpec((1, CLS), lambda i: (0, 0)),
        ],
        out_specs=pl.BlockSpec((G, CLS), lambda i: (0, 0)),
        out_shape=jax.ShapeDtypeStruct((G, CLS), jnp.float32),
        scratch_shapes=[pltpu.VMEM((G, 128), jnp.float32)],
    )(*raws, b, w, batch, fcw, fcb)


def _vgather(v, idx):
    dnums = lax.GatherDimensionNumbers(
        offset_dims=(), collapsed_slice_dims=(0,), start_index_map=(0,))
    return lax.gather(v, idx[:, None], dnums, (1,),
                      mode=lax.GatherScatterMode.PROMISE_IN_BOUNDS)


def _make_edge_kernel(msgw, nbkt, nb, flc):
    nsl = 1 if msgw == 64 else 5
    mrows = CE if msgw == 64 else 64
    halves = CE // mrows
    nfl = nb // flc
    mesh = plsc.VectorSubcoreMesh(core_axis_name="c", subcore_axis_name="s")
    cp = pltpu.CompilerParams()
    if "needs_layout_passes" in pltpu.CompilerParams.__dataclass_fields__:
        cp = dataclasses.replace(cp, needs_layout_passes=False)

    @functools.partial(
        pl.kernel, mesh=mesh, compiler_params=cp,
        out_type=[jax.ShapeDtypeStruct((NP, 128), jnp.float32)
                  for _ in range(nsl)],
        scratch_types=[
            pltpu.VMEM((CE,), jnp.int32),
            pltpu.VMEM((CE,), jnp.int32),
            pltpu.VMEM((mrows,), jnp.int32),
            pltpu.VMEM((CE, 128), jnp.float32),
            pltpu.VMEM((8, nb), jnp.float32),
            pltpu.VMEM((8, 16), jnp.float32),
            pltpu.VMEM((flc, 128), jnp.float32),
            pltpu.VMEM((1, 16), jnp.float32),
            pltpu.VMEM((32,), jnp.int32),
        ] + [pltpu.VMEM((mrows, 128), jnp.float32) for _ in range(nsl)]
          + [pltpu.VMEM_SHARED((nb + 8, 128), jnp.float32)
             for _ in range(nsl)])
    def k(src_hbm, dst_hbm, starts_hbm, attt_hbm, fx_hbm, c_hbm, *rest):
        outs = rest[:nsl]
        (sVM, dVM, dloc, gath, adstT, exT, zbo, cvm, starts) = \
            rest[nsl:nsl + 9]
        msgbs = rest[nsl + 9:nsl + 9 + nsl]
        slabs = rest[nsl + 9 + nsl:]
        cid = lax.axis_index("c")
        sid = lax.axis_index("s")
        pltpu.sync_copy(starts_hbm, starts)
        pltpu.sync_copy(c_hbm, cvm)
        i16 = lax.iota(jnp.int32, 16)
        sv1 = starts[pl.ds(0, 16)]
        sv2 = starts[pl.ds(16, 16)]
        c16 = cvm[0, :]
        lanelt8 = i16 < 8
        z16 = jnp.zeros((16,), jnp.float32)

        def sget(kk):
            return (jnp.sum(jnp.where(i16 == kk, sv1, 0), axis=0)
                    + jnp.sum(jnp.where(i16 + 16 == kk, sv2, 0), axis=0))

        @pl.loop(0, flc)
        def _(r):
            @pl.loop(0, 128, step=16)
            def _(c2):
                zbo[r, pl.ds(c2, 16)] = z16

        for mb in msgbs:
            @pl.loop(0, mrows)
            def _(r):
                @pl.loop(0, 128, step=16)
                def _(c2):
                    mb[r, pl.ds(c2, 16)] = z16

        @pl.loop(0, nbkt)
        def _(b):
            bkt = cid * nbkt + b
            base = bkt * nb
            s_lo = sget(bkt)
            s_hi = sget(bkt + 1)
            sa = (s_lo // 128) * 128

            pltpu.sync_copy(attt_hbm.at[pl.ds(8, 8), pl.ds(base, nb)], adstT)

            for sl in slabs:
                @pl.loop(sid, nfl, step=16)
                def _(j):
                    pltpu.sync_copy(zbo, sl.at[pl.ds(j * flc, flc)])

            plsc.subcore_barrier()

            span = s_hi - sa
            nt = (span + (16 * CE - 1)) // (16 * CE)

            @pl.loop(0, nt)
            def _(t):
                off = sa + (t * 16 + sid) * CE
                pltpu.sync_copy(src_hbm.at[pl.ds(off, CE)], sVM)
                pltpu.sync_copy(dst_hbm.at[pl.ds(off, CE)], dVM)
                pltpu.sync_copy(fx_hbm.at[sVM], gath)

                for hf in range(halves):
                    @pl.loop(hf * mrows, (hf + 1) * mrows, step=16)
                    def _(kk):
                        m0 = kk - hf * mrows
                        d16 = dVM[pl.ds(kk, 16)]
                        idx16 = i16 + (off + kk)
                        valid = (idx16 >= s_lo) & (idx16 < s_hi)
                        dloc16 = jnp.where(valid, d16 - base, nb)
                        dloc[pl.ds(m0, 16)] = dloc16
                        dg = jnp.minimum(dloc16, nb - 1)
                        rowv = i16 + kk
                        for hh in range(8):
                            asr = plsc.load_gather(
                                gath, [rowv, i16 * 0 + 64 + hh])
                            ads = plsc.load_gather(adstT, [i16 * 0 + hh, dg])
                            al = asr + ads
                            al = jnp.where(al >= 0, al, 0.2 * al)
                            exv = jnp.exp(al - _vgather(c16, i16 * 0 + hh))
                            exT[hh, :] = exv
                        for el in range(16):
                            e = kk + el
                            r = m0 + el
                            coefA = plsc.load_gather(
                                exT, [i16 & 7, i16 * 0 + el])
                            exrow = jnp.where(lanelt8, coefA, 0.0)
                            if msgw == 64:
                                msgbs[0][r, pl.ds(64, 16)] = exrow
                                for j in range(4):
                                    cj = _vgather(coefA, (i16 >> 3) + 2 * j)
                                    msgbs[0][r, pl.ds(16 * j, 16)] = (
                                        gath[e, pl.ds(16 * j, 16)] * cj)
                            else:
                                msgbs[4][r, pl.ds(0, 16)] = exrow
                                f4 = [gath[e, pl.ds(16 * j, 16)]
                                      for j in range(4)]
                                for hh in range(8):
                                    ch = _vgather(coefA, i16 * 0 + hh)
                                    for j in range(4):
                                        c0 = 64 * hh + 16 * j
                                        msgbs[c0 // 128][r, pl.ds(
                                            c0 % 128, 16)] = f4[j] * ch

                    for mb, sl in zip(msgbs, slabs):
                        pltpu.sync_copy(mb, sl.at[dloc], add=True)

            plsc.subcore_barrier()

            for sl, oh in zip(slabs, outs):
                @pl.loop(sid, nfl, step=16)
                def _(j):
                    pltpu.sync_copy(sl.at[pl.ds(j * flc, flc)],
                                    oh.at[pl.ds(base + j * flc, flc)])

            plsc.subcore_barrier()

    return k


@functools.lru_cache(maxsize=None)
def _edge_kernels():
    return (_make_edge_kernel(64, 1, 5120, 40),
            _make_edge_kernel(512, 8, 640, 8))


def _blockdiag(att):
    hh, cc = att.shape
    return (att[:, :, None] * jnp.eye(hh, dtype=att.dtype)[:, None, :]
            ).reshape(hh * cc, hh)


def kernel(x, edge_index, batch, W1, as1, ad1, b1, W2, as2, ad2, b2,
           W3, as3, ad3, b3, fcW, fcb):
    src = edge_index[0].astype(jnp.int32)
    dst = edge_index[1].astype(jnp.int32)
    q = jnp.sort(dst * 16384 + src)
    dstS = (q >> 14).astype(jnp.int32)
    srcS = jnp.bitwise_and(q, 16383).astype(jnp.int32)
    zpad = jnp.zeros((EPAD - E,), jnp.int32)
    srcP = jnp.concatenate([srcS, zpad])
    dstP = jnp.concatenate([dstS, zpad])

    cuts = jnp.searchsorted(
        dstS, jnp.arange(0, NP + 640, 640, dtype=jnp.int32)
    ).astype(jnp.int32)
    starts2 = jnp.concatenate([cuts[0:17:8], jnp.full((29,), E, jnp.int32)])
    starts4 = jnp.concatenate([cuts[0:17], jnp.full((15,), E, jnp.int32)])

    a1 = jnp.concatenate([_blockdiag(as1[0]), _blockdiag(ad1[0])], axis=1)
    a2 = jnp.concatenate([_blockdiag(as2[0]), _blockdiag(ad2[0])], axis=1)
    w3r = W3.reshape(64, 8, 64)
    v3 = jnp.concatenate([jnp.einsum('dhc,hc->dh', w3r, as3[0]),
                          jnp.einsum('dhc,hc->dh', w3r, ad3[0])], axis=1)
    w3s = w3r.transpose(1, 0, 2).reshape(512, 64)
    eye64 = jnp.eye(64, dtype=jnp.float32)

    xp = jnp.pad(x.astype(jnp.float32), ((0, NP - N), (0, 0)))
    batchp = jnp.pad(batch.astype(jnp.int32), (0, NP - N),
                     constant_values=G)

    edge12, edge3 = _edge_kernels()
    fx1, attt1, c1 = _tc_first(xp, W1, a1)
    [raw1] = edge12(srcP, dstP, starts2, attt1, fx1, c1)
    fx2, attt2, c2 = _tc_mid(raw1, b1.reshape(1, 64), W2, a2)
    [raw2] = edge12(srcP, dstP, starts2, attt2, fx2, c2)
    fx3, attt3, c3 = _tc_mid(raw2, b2.reshape(1, 64), eye64, v3)
    raw3 = edge3(srcP, dstP, starts4, attt3, fx3, c3)
    return _tc_out(raw3, b3.reshape(1, 64), w3s,
                   batchp.reshape(NBLK, 1, BLK), fcW, fcb.reshape(1, CLS))
